# Initial kernel scaffold; baseline (speedup 1.0000x reference)
import jax
import jax.numpy as jnp
from jax.experimental import pallas as pl


def kernel(trace_all, x, W_lin, W_lin2, fc1_w, fc1_b, fc2_w, fc2_b, W_enc1, W_enc2, Wc, bc, edge_index, slow_edge_mask, insert_node_mask, gumbel_noise):
    raise NotImplementedError("write your pallas kernel here")


# speedup vs baseline: 2.4804x; 2.4804x over previous
import functools

import jax
import jax.numpy as jnp
from jax import lax
from jax.experimental import pallas as pl
from jax.experimental.pallas import tpu as pltpu
from jax.experimental.pallas import tpu_sc as plsc

_N = 10000
_NE = 106667
_E = 3 * _NE
_D = 128
_H = 128
_OUT = 70

_BLK = 8192
_NEP = 114688
_EP = 3 * _NEP
_NR = 10240
_TRASH = 10000
_W1 = 144
_W2 = 128

_NW = 32
_EB = 128
_SCORE_BLKS = _NEP // (_NW * _EB)
_SEG_BLKS = _EP // (_NW * _EB)
_SEG1_BLKS = _EP // (16 * _EB)
_RPS = _NR // 16
_WBR = 160


def _node_tables_body(ta, wlin, wlin2, fc1w, fc1b, fc2wt, p12_o, v_o):
    t = jnp.concatenate([ta[0], ta[1]], axis=1)
    h = jnp.maximum(jnp.dot(t, wlin[...], preferred_element_type=jnp.float32), 0.0)
    m = jnp.dot(h, wlin2[...], preferred_element_type=jnp.float32)
    nrm = jnp.sqrt(jnp.sum(m * m, axis=1, keepdims=True))
    mvc = m / jnp.maximum(nrm, 1e-12)
    p1 = (jnp.dot(mvc, fc1w[:128], preferred_element_type=jnp.float32)
          + fc1b[0])
    p2 = jnp.dot(mvc, fc1w[128:], preferred_element_type=jnp.float32)
    p12_o[...] = jnp.concatenate([p1, p2], axis=1)
    v_o[...] = jnp.broadcast_to(fc2wt[0] - fc2wt[1], (8, 64))


def _prep_body(g0, g1, fsrc, fdst, fc2b, thr_o, src_o, dst_o):
    b = pl.program_id(0)
    p = b * _BLK + lax.broadcasted_iota(jnp.int32, (1, _BLK), 1)
    inr = p < _NE
    c = fc2b[0, 0] - fc2b[0, 1]
    thr_o[...] = jnp.where(inr, g1[...] - g0[...] - c, jnp.float32(1e30))
    src_o[...] = jnp.where(inr, fsrc[...], 0)
    dst_o[...] = jnp.where(inr, fdst[...], 0)


def _dst_body(s, d, acc16, thr, src_o, dst_o):
    k = pl.program_id(0)
    b = pl.program_id(1)
    p = b * _BLK + lax.broadcasted_iota(jnp.int32, (1, _BLK), 1)
    inr = p < _NE
    tot = jnp.sum(acc16[...], axis=1).reshape(1, _BLK)
    av = jnp.where(tot >= thr[...], 1, 0)
    sel = jnp.where(k == 0, av, 1 - av)
    keep = jnp.logical_and(inr, sel > 0)
    src_o[...] = jnp.where(inr, s[0], 0).reshape(1, 1, _BLK)
    dst_o[...] = jnp.where(keep, d[0], _TRASH).reshape(1, 1, _BLK)


def _layer1_body(feat, deg128, wenc1, h_o, deg_o):
    deg = jnp.maximum(deg128[:, :1], 1.0)
    agg = feat[...] / deg
    h_o[...] = jnp.maximum(
        jnp.dot(agg, wenc1[...], preferred_element_type=jnp.float32), 0.0)
    deg_o[...] = jnp.broadcast_to(deg, (deg.shape[0], 8))


def _layer2_body(pa, pb, degc, wenc2, wc, bc, h_o, log_o):
    s = pa[...] + pb[...]
    agg = s / degc[:, :1]
    h = jnp.maximum(
        jnp.dot(agg, wenc2[...], preferred_element_type=jnp.float32), 0.0)
    h_o[...] = h
    log_o[...] = jnp.dot(h, wc[...], preferred_element_type=jnp.float32) + bc[0]


def _score_body(p12, fsrc, fdst, vpad, acc_o,
                idx_s, idx_d, rs, rt, acc2d, vbuf, sem1, sem2):
    c = lax.axis_index("c")
    s = lax.axis_index("s")
    wid = s * 2 + c
    pltpu.sync_copy(vpad.at[0], vbuf)
    v0 = vbuf[pl.ds(0, 16)]
    v1 = vbuf[pl.ds(16, 16)]
    v2 = vbuf[pl.ds(32, 16)]
    v3 = vbuf[pl.ds(48, 16)]

    def blk(j, _):
        off = (wid * _SCORE_BLKS + j) * _EB
        pltpu.sync_copy(fsrc.at[pl.ds(off, _EB)], idx_s)
        pltpu.sync_copy(fdst.at[pl.ds(off, _EB)], idx_d)
        d1 = pltpu.async_copy(p12.at[idx_s], rs, sem1)
        d2 = pltpu.async_copy(p12.at[idx_d], rt, sem2)
        d1.wait()
        d2.wait()

        def edge(e, _):
            a0 = jnp.maximum(rs[e, pl.ds(0, 16)] + rt[e, pl.ds(64, 16)], 0.0)
            a1 = jnp.maximum(rs[e, pl.ds(16, 16)] + rt[e, pl.ds(80, 16)], 0.0)
            a2 = jnp.maximum(rs[e, pl.ds(32, 16)] + rt[e, pl.ds(96, 16)], 0.0)
            a3 = jnp.maximum(rs[e, pl.ds(48, 16)] + rt[e, pl.ds(112, 16)], 0.0)
            acc2d[e, pl.ds(0, 16)] = a0 * v0 + a1 * v1 + a2 * v2 + a3 * v3
            return 0

        lax.fori_loop(0, _EB, edge, 0)
        pltpu.sync_copy(acc2d, acc_o.at[pl.ds(off, _EB)])
        return 0

    lax.fori_loop(0, _SCORE_BLKS, blk, 0)


def _zero_rows(bounce, acc, base_row):
    def zr(r, _):
        for k in range(8):
            bounce[r, pl.ds(16 * k, 16)] = jnp.zeros((16,), jnp.float32)
        return 0

    lax.fori_loop(0, _WBR, zr, 0)
    for q in range(_RPS // _WBR):
        pltpu.sync_copy(bounce, acc.at[pl.ds(base_row + q * _WBR, _WBR)])


def _seg1_body(table, srcp, dstp, feat_o, deg_o,
               idx_s, idx_d, rows, bounce, acc, sem):
    c = lax.axis_index("c")
    s = lax.axis_index("s")
    base_row = s * _RPS
    _zero_rows(bounce, acc, base_row)
    plsc.subcore_barrier()

    @pl.when(c == 0)
    def _():
        def eb(j, _):
            off = (s * _SEG1_BLKS + j) * _EB
            pltpu.sync_copy(srcp.at[pl.ds(off, _EB)], idx_s)
            pltpu.sync_copy(dstp.at[pl.ds(off, _EB)], idx_d)
            pltpu.async_copy(table.at[idx_s], rows, sem).wait()
            pltpu.sync_copy(rows, acc.at[idx_d], add=True)
            return 0

        lax.fori_loop(0, _SEG1_BLKS, eb, 0)

    @pl.when(c == 1)
    def _():
        def fill(r, _):
            for k in range(8):
                rows[r, pl.ds(16 * k, 16)] = jnp.ones((16,), jnp.float32)
            return 0

        lax.fori_loop(0, _EB, fill, 0)

        def eb(j, _):
            off = (s * _SEG1_BLKS + j) * _EB
            pltpu.sync_copy(dstp.at[pl.ds(off, _EB)], idx_d)
            pltpu.sync_copy(rows, acc.at[idx_d], add=True)
            return 0

        lax.fori_loop(0, _SEG1_BLKS, eb, 0)

    plsc.subcore_barrier()
    for h in range(_RPS // _WBR):
        row = base_row + h * _WBR
        pltpu.sync_copy(acc.at[pl.ds(row, _WBR)], bounce)

        @pl.when(c == 0)
        def _():
            pltpu.sync_copy(bounce, feat_o.at[pl.ds(row, _WBR)])

        @pl.when(c == 1)
        def _():
            pltpu.sync_copy(bounce, deg_o.at[pl.ds(row, _WBR)])


def _seg2_body(table, srcp, dstp, out0, out1,
               idx_s, idx_d, rows, bounce, acc, sem):
    c = lax.axis_index("c")
    s = lax.axis_index("s")
    wid = s * 2 + c
    base_row = s * _RPS
    _zero_rows(bounce, acc, base_row)
    plsc.subcore_barrier()

    def eb(j, _):
        off = (wid * _SEG_BLKS + j) * _EB
        pltpu.sync_copy(srcp.at[pl.ds(off, _EB)], idx_s)
        pltpu.sync_copy(dstp.at[pl.ds(off, _EB)], idx_d)
        pltpu.async_copy(table.at[idx_s], rows, sem).wait()
        pltpu.sync_copy(rows, acc.at[idx_d], add=True)
        return 0

    lax.fori_loop(0, _SEG_BLKS, eb, 0)
    plsc.subcore_barrier()

    for h in range(_RPS // _WBR):
        row = base_row + h * _WBR
        pltpu.sync_copy(acc.at[pl.ds(row, _WBR)], bounce)

        @pl.when(c == 0)
        def _():
            pltpu.sync_copy(bounce, out0.at[pl.ds(row, _WBR)])

        @pl.when(c == 1)
        def _():
            pltpu.sync_copy(bounce, out1.at[pl.ds(row, _WBR)])


def _sc_mesh():
    return plsc.VectorSubcoreMesh(core_axis_name="c", subcore_axis_name="s")


_SC_PARAMS = pltpu.CompilerParams(use_tc_tiling_on_sc=False)


@functools.partial(jax.jit, donate_argnums=())
def kernel(trace_all, x, W_lin, W_lin2, fc1_w, fc1_b, fc2_w, fc2_b,
           W_enc1, W_enc2, Wc, bc, edge_index, slow_edge_mask,
           insert_node_mask, gumbel_noise):
    del slow_edge_mask, insert_node_mask
    f32 = jnp.float32
    i32 = jnp.int32
    bn = 1000

    p12_t, vpad = pl.pallas_call(
        _node_tables_body,
        grid=(_N // bn,),
        in_specs=[
            pl.BlockSpec((2, bn, _H), lambda i: (0, i, 0)),
            pl.BlockSpec((256, 128), lambda i: (0, 0)),
            pl.BlockSpec((128, 128), lambda i: (0, 0)),
            pl.BlockSpec((256, 64), lambda i: (0, 0)),
            pl.BlockSpec((1, 64), lambda i: (0, 0)),
            pl.BlockSpec((2, 64), lambda i: (0, 0)),
        ],
        out_specs=[
            pl.BlockSpec((bn, 128), lambda i: (i, 0)),
            pl.BlockSpec((8, 64), lambda i: (0, 0)),
        ],
        out_shape=[
            jax.ShapeDtypeStruct((_N, 128), f32),
            jax.ShapeDtypeStruct((8, 64), f32),
        ],
    )(trace_all, W_lin, W_lin2, fc1_w, fc1_b.reshape(1, 64), fc2_w.T)

    src_r = edge_index[0].reshape(3, _NE)
    dst_r = edge_index[1].reshape(3, _NE)
    g0 = gumbel_noise[:, 0].reshape(1, _NE)
    g1 = gumbel_noise[:, 1].reshape(1, _NE)
    thr, fsrc, fdst = pl.pallas_call(
        _prep_body,
        grid=(_NEP // _BLK,),
        in_specs=[
            pl.BlockSpec((1, _BLK), lambda b: (0, b)),
            pl.BlockSpec((1, _BLK), lambda b: (0, b)),
            pl.BlockSpec((1, _BLK), lambda b: (0, b)),
            pl.BlockSpec((1, _BLK), lambda b: (0, b)),
            pl.BlockSpec((1, 2), lambda b: (0, 0)),
        ],
        out_specs=[
            pl.BlockSpec((1, _BLK), lambda b: (0, b)),
            pl.BlockSpec((1, _BLK), lambda b: (0, b)),
            pl.BlockSpec((1, _BLK), lambda b: (0, b)),
        ],
        out_shape=[
            jax.ShapeDtypeStruct((1, _NEP), f32),
            jax.ShapeDtypeStruct((1, _NEP), i32),
            jax.ShapeDtypeStruct((1, _NEP), i32),
        ],
    )(g0, g1, src_r[0:1], dst_r[0:1], fc2_b.reshape(1, 2))

    score = pl.kernel(
        _score_body,
        out_type=jax.ShapeDtypeStruct((_NEP, 16), f32),
        mesh=_sc_mesh(),
        scratch_types=[
            pltpu.VMEM((_EB,), i32),
            pltpu.VMEM((_EB,), i32),
            pltpu.VMEM((_EB, 128), f32),
            pltpu.VMEM((_EB, 128), f32),
            pltpu.VMEM((_EB, 16), f32),
            pltpu.VMEM((64,), f32),
            pltpu.SemaphoreType.DMA,
            pltpu.SemaphoreType.DMA,
        ],
        compiler_params=_SC_PARAMS,
    )
    acc16 = score(p12_t, fsrc.reshape(_NEP), fdst.reshape(_NEP), vpad)

    srcp, dstp = pl.pallas_call(
        _dst_body,
        grid=(3, _NEP // _BLK),
        in_specs=[
            pl.BlockSpec((1, 1, _BLK), lambda k, b: (k, 0, b)),
            pl.BlockSpec((1, 1, _BLK), lambda k, b: (k, 0, b)),
            pl.BlockSpec((_BLK, 16), lambda k, b: (b, 0)),
            pl.BlockSpec((1, _BLK), lambda k, b: (0, b)),
        ],
        out_specs=[
            pl.BlockSpec((1, 1, _BLK), lambda k, b: (k, 0, b)),
            pl.BlockSpec((1, 1, _BLK), lambda k, b: (k, 0, b)),
        ],
        out_shape=[
            jax.ShapeDtypeStruct((3, 1, _NEP), i32),
            jax.ShapeDtypeStruct((3, 1, _NEP), i32),
        ],
    )(src_r.reshape(3, 1, _NE), dst_r.reshape(3, 1, _NE), acc16, thr)
    srcp = srcp.reshape(_EP)
    dstp = dstp.reshape(_EP)

    seg1 = pl.kernel(
        _seg1_body,
        out_type=(jax.ShapeDtypeStruct((_NR, _W2), f32),
                  jax.ShapeDtypeStruct((_NR, _W2), f32)),
        mesh=_sc_mesh(),
        scratch_types=[
            pltpu.VMEM((_EB,), i32),
            pltpu.VMEM((_EB,), i32),
            pltpu.VMEM((_EB, _W2), f32),
            pltpu.VMEM((_WBR, _W2), f32),
            pltpu.VMEM_SHARED((_NR, _W2), f32),
            pltpu.SemaphoreType.DMA,
        ],
        compiler_params=_SC_PARAMS,
    )
    feat1, deg128 = seg1(x, srcp, dstp)

    h1, degc = pl.pallas_call(
        _layer1_body,
        grid=(_NR // 640,),
        in_specs=[
            pl.BlockSpec((640, _W2), lambda i: (i, 0)),
            pl.BlockSpec((640, _W2), lambda i: (i, 0)),
            pl.BlockSpec((128, 128), lambda i: (0, 0)),
        ],
        out_specs=[
            pl.BlockSpec((640, _H), lambda i: (i, 0)),
            pl.BlockSpec((640, 8), lambda i: (i, 0)),
        ],
        out_shape=[
            jax.ShapeDtypeStruct((_N, _H), f32),
            jax.ShapeDtypeStruct((_N, 8), f32),
        ],
    )(feat1, deg128, W_enc1)

    seg2 = pl.kernel(
        _seg2_body,
        out_type=(jax.ShapeDtypeStruct((_NR, _W2), f32),
                  jax.ShapeDtypeStruct((_NR, _W2), f32)),
        mesh=_sc_mesh(),
        scratch_types=[
            pltpu.VMEM((_EB,), i32),
            pltpu.VMEM((_EB,), i32),
            pltpu.VMEM((_EB, _W2), f32),
            pltpu.VMEM((_WBR, _W2), f32),
            pltpu.VMEM_SHARED((_NR, _W2), f32),
            pltpu.SemaphoreType.DMA,
        ],
        compiler_params=_SC_PARAMS,
    )
    part2a, part2b = seg2(h1, srcp, dstp)

    h2, logits = pl.pallas_call(
        _layer2_body,
        grid=(_NR // 640,),
        in_specs=[
            pl.BlockSpec((640, _W2), lambda i: (i, 0)),
            pl.BlockSpec((640, _W2), lambda i: (i, 0)),
            pl.BlockSpec((640, 8), lambda i: (i, 0)),
            pl.BlockSpec((128, 128), lambda i: (0, 0)),
            pl.BlockSpec((128, _OUT), lambda i: (0, 0)),
            pl.BlockSpec((1, _OUT), lambda i: (0, 0)),
        ],
        out_specs=[
            pl.BlockSpec((640, _H), lambda i: (i, 0)),
            pl.BlockSpec((640, _OUT), lambda i: (i, 0)),
        ],
        out_shape=[
            jax.ShapeDtypeStruct((_N, _H), f32),
            jax.ShapeDtypeStruct((_N, _OUT), f32),
        ],
    )(part2a, part2b, degc, W_enc2, Wc, bc.reshape(1, _OUT))

    trace_out = jnp.stack([h1, h2], axis=0)
    return (logits, trace_out)
